# baseline (speedup 1.0000x reference)
import functools

import jax
import jax.numpy as jnp
from jax import lax
from jax.experimental import pallas as pl
from jax.experimental.pallas import tpu as pltpu
from jax.experimental.pallas import tpu_sc as plsc

K = 16
LF = 4
NFR = LF - 1
NS = 24
NBATCH = 8
RSC = 256
RMLP = 256
CH = 384
INT_BIG = 2**31 - 1


AB = 128
CHK = 16


def _xor_perm(x, j, ri):
    up = jnp.roll(x, -j, axis=0)
    dn = jnp.roll(x, j, axis=0)
    return jnp.where((ri & j) == 0, up, dn)


def _bitonic_stage(v, ix, j, take_min, ri):
    pv = _xor_perm(v, j, ri)
    pi = _xor_perm(ix, j, ri)
    swap = v > pv
    mnv = jnp.where(swap, pv, v)
    mni = jnp.where(swap, pi, ix)
    mxv = jnp.where(swap, v, pv)
    mxi = jnp.where(swap, ix, pi)
    return (jnp.where(take_min, mnv, mxv),
            jnp.where(take_min, mni, mxi))


def _sort_desc(v, ix, ri, size):
    k = 2
    while k <= size:
        upmask = (ri & k) == 0
        j = k // 2
        while j >= 1:
            jm = (ri & j) == 0
            take_min = ~(jm == upmask)
            v, ix = _bitonic_stage(v, ix, j, take_min, ri)
            j //= 2
        k *= 2
    return v, ix


NHALF = 4
ABW = AB * NHALF


def _knn_body(anch_ref, cand_ref, rs_ref, re_ref, dr_ref, idx_ref):
    ax, ay, az, rs, re = [], [], [], [], []
    for hb in range(NHALF):
        sl = slice(hb * AB, (hb + 1) * AB)
        ax.append(anch_ref[0:1, sl])
        ay.append(anch_ref[1:2, sl])
        az.append(anch_ref[2:3, sl])
        rs.append(rs_ref[0:1, sl])
        re.append(re_ref[0:1, sl])
    lo = jnp.min(rs_ref[...]) // CHK
    hi = (jnp.max(re_ref[...]) + CHK - 1) // CHK
    ri = lax.broadcasted_iota(jnp.int32, (CHK, 1), 0)

    sv0 = jnp.full((CHK, AB), jnp.inf, dtype=jnp.float32)
    si0 = jnp.broadcast_to(2**30 + ri, (CHK, AB))

    def chunk_step(c, carry):
        base = c * CHK
        row = base + ri
        out = []
        for f in range(NFR):
            ch = cand_ref[f, pl.ds(base, CHK), :]
            cx = ch[:, 0:1]
            cy = ch[:, 1:2]
            cz = ch[:, 2:3]
            for hb in range(NHALF):
                sv, si = carry[2 * (NHALF * f + hb)], \
                    carry[2 * (NHALF * f + hb) + 1]
                dx = cx - ax[hb]
                dy = cy - ay[hb]
                dz = cz - az[hb]
                d = dx * dx + dy * dy + dz * dz
                valid = (row >= rs[hb]) & (row < re[hb])
                d = jnp.where(valid, d, jnp.inf)
                cix = jnp.broadcast_to(row, (CHK, AB))
                cv, cix = _sort_desc(d, cix, ri, CHK)
                swap = sv > cv
                v = jnp.where(swap, cv, sv)
                ix = jnp.where(swap, cix, si)
                for j in (8, 4, 2, 1):
                    v, ix = _bitonic_stage(v, ix, j, (ri & j) == 0, ri)
                out += [v, ix]
        return tuple(out)

    init = (sv0, si0) * (NFR * NHALF)
    res = lax.fori_loop(lo, hi, chunk_step, init)
    for f in range(NFR):
        dr_ref[f] = jnp.sqrt(jnp.concatenate(
            [res[2 * (NHALF * f + hb)] for hb in range(NHALF)], axis=1))
        idx_ref[f] = jnp.concatenate(
            [res[2 * (NHALF * f + hb) + 1] for hb in range(NHALF)], axis=1)


def _run_knn(anchT, xyz_frames, row_start1, row_end1, n):
    grid = (n // ABW,)
    return pl.pallas_call(
        _knn_body,
        grid=grid,
        in_specs=[
            pl.BlockSpec((3, ABW), lambda ab: (0, ab)),
            pl.BlockSpec((NFR, n, 3), lambda ab: (0, 0, 0)),
            pl.BlockSpec((1, ABW), lambda ab: (0, ab)),
            pl.BlockSpec((1, ABW), lambda ab: (0, ab)),
        ],
        out_specs=[
            pl.BlockSpec((NFR, CHK, ABW), lambda ab: (0, 0, ab)),
            pl.BlockSpec((NFR, CHK, ABW), lambda ab: (0, 0, ab)),
        ],
        out_shape=[
            jax.ShapeDtypeStruct((NFR, CHK, n), jnp.float32),
            jax.ShapeDtypeStruct((NFR, CHK, n), jnp.int32),
        ],
    )(anchT, xyz_frames, row_start1, row_end1)


def _score_body(dr_ref, idx_ref, g_ref, prt_ref, rid_ref, *, n):
    cols = NFR * K
    x = dr_ref[...] * (-0.5)
    m = jnp.max(x, axis=0, keepdims=True)
    e = jnp.exp(x - m)
    pr_r = e / jnp.sum(e, axis=0, keepdims=True)
    p = 0.5 * pr_r + 0.5 * prt_ref[...]
    sc = jnp.log(p + 1e-12) + g_ref[...]

    pid = pl.program_id(0)
    ri48 = lax.broadcasted_iota(jnp.int32, (cols, 1), 0)
    li = lax.broadcasted_iota(jnp.int32, (1, AB), 1)
    pos = cols * (pid * AB + li) + ri48
    frame = pos // (n * K)
    nbg = idx_ref[...] + frame * n
    nbg = jnp.minimum(nbg, jnp.int32(NFR * n - 1))
    comb = ri48 * 32768 + nbg

    pad = 64 - cols
    scp = jnp.concatenate(
        [sc, jnp.full((pad, AB), -jnp.inf, dtype=jnp.float32)], axis=0)
    cbp = jnp.concatenate(
        [comb, jnp.full((pad, AB), INT_BIG, dtype=jnp.int32)], axis=0)
    ri64 = lax.broadcasted_iota(jnp.int32, (64, 1), 0)
    scp, cbp = _sort_desc(scp, cbp, ri64, 64)
    rid_ref[...] = cbp[0:NS, :] & 32767


def _run_score(drT_scr, idxT_scr, gT, prtT, n):
    cols = NFR * K
    grid = (n // AB,)
    return pl.pallas_call(
        functools.partial(_score_body, n=n),
        grid=grid,
        in_specs=[
            pl.BlockSpec((cols, AB), lambda rb: (0, rb)),
            pl.BlockSpec((cols, AB), lambda rb: (0, rb)),
            pl.BlockSpec((cols, AB), lambda rb: (0, rb)),
            pl.BlockSpec((cols, 1), lambda rb: (0, 0)),
        ],
        out_specs=pl.BlockSpec((NS, AB), lambda rb: (0, rb)),
        out_shape=jax.ShapeDtypeStruct((NS, n), jnp.int32),
    )(drT_scr, idxT_scr, gT, prtT)


def _proj_body(f_ref, w1_ref, b1_ref, out_ref, *, h, nblk_src):
    pid = pl.program_id(0)
    w1a = w1_ref[0:h, :]
    w1b = w1_ref[h:2 * h, :]

    @pl.when(pid < nblk_src)
    def _():
        out_ref[...] = jnp.dot(f_ref[...], w1b,
                               preferred_element_type=jnp.float32)

    @pl.when(pid >= nblk_src)
    def _():
        out_ref[...] = (jnp.dot(f_ref[...], w1a + w1b,
                                preferred_element_type=jnp.float32)
                        + b1_ref[...])


def _run_proj(f_all, w1, b1_row, n, h):
    rows = LF * n
    blk = 512
    nblk_src = (NFR * n) // blk
    return pl.pallas_call(
        functools.partial(_proj_body, h=h, nblk_src=nblk_src),
        grid=(rows // blk,),
        in_specs=[
            pl.BlockSpec((blk, h), lambda rb: (rb, 0)),
            pl.BlockSpec((2 * h, h), lambda rb: (0, 0)),
            pl.BlockSpec((1, h), lambda rb: (0, 0)),
        ],
        out_specs=pl.BlockSpec((blk, h), lambda rb: (rb, 0)),
        out_shape=jax.ShapeDtypeStruct((rows, h), jnp.float32),
    )(f_all, w1, b1_row)


def _run_sc_gather(table, rid_flat, n, h):
    total = n * NS
    info = plsc.get_sparse_core_info()
    nw = info.num_cores * info.num_subcores
    b_per_w = total // nw
    nch = b_per_w // CH
    mesh = plsc.VectorSubcoreMesh(core_axis_name="c", subcore_axis_name="s")

    @functools.partial(
        pl.kernel,
        mesh=mesh,
        out_type=jax.ShapeDtypeStruct((total, h), jnp.float32),
        scratch_types=[
            pltpu.VMEM((CH,), jnp.int32),
            pltpu.VMEM((CH,), jnp.int32),
            pltpu.VMEM((CH, h), jnp.float32),
            pltpu.VMEM((CH, h), jnp.float32),
            pltpu.SemaphoreType.DMA,
            pltpu.SemaphoreType.DMA,
        ],
    )
    def gather_k(table_hbm, idx_hbm, out_hbm, idx_v0, idx_v1,
                 rows_v0, rows_v1, sem0, sem1):
        wid = lax.axis_index("s") * info.num_cores + lax.axis_index("c")
        base = wid * b_per_w
        idxs = [idx_v0, idx_v1]
        rows = [rows_v0, rows_v1]
        sems = [sem0, sem1]

        pltpu.sync_copy(idx_hbm.at[pl.ds(base, CH)], idxs[0])
        handles = [pltpu.async_copy(table_hbm.at[idxs[0]], rows[0], sems[0]),
                   None]
        for c in range(nch):
            b = c % 2
            nb = (c + 1) % 2
            if c + 1 < nch:
                off_n = base + (c + 1) * CH
                pltpu.sync_copy(idx_hbm.at[pl.ds(off_n, CH)], idxs[nb])
                handles[nb] = pltpu.async_copy(
                    table_hbm.at[idxs[nb]], rows[nb], sems[nb])
            handles[b].wait()
            pltpu.sync_copy(rows[b], out_hbm.at[pl.ds(base + c * CH, CH)])

    return gather_k(table, rid_flat)


def _tail_body(a_ref, b_ref, w2_ref, b2_ref, out_ref):
    a = a_ref[...]
    w2 = w2_ref[...]
    acc = None
    for s in range(NS):
        hsd = jnp.maximum(a - b_ref[:, s, :], 0.0)
        o = jnp.dot(hsd, w2, preferred_element_type=jnp.float32)
        acc = o if acc is None else jnp.maximum(acc, o)
    out_ref[...] = jnp.tanh(acc + b2_ref[...])


def _run_tail(a_proj, b_rows, w2, b2_row, n, h):
    return pl.pallas_call(
        _tail_body,
        grid=(n // RMLP,),
        in_specs=[
            pl.BlockSpec((RMLP, h), lambda rb: (rb, 0)),
            pl.BlockSpec((RMLP, NS, h), lambda rb: (rb, 0, 0)),
            pl.BlockSpec((h, h), lambda rb: (0, 0)),
            pl.BlockSpec((1, h), lambda rb: (0, 0)),
        ],
        out_specs=pl.BlockSpec((RMLP, h), lambda rb: (rb, 0)),
        out_shape=jax.ShapeDtypeStruct((n, h), jnp.float32),
    )(a_proj, b_rows, w2, b2_row)


def kernel(num_samples, f1_list, xyz1_list, dbatch, W1, b1, W2, b2):
    L, n, h = f1_list.shape
    del num_samples

    anchT = jnp.transpose(xyz1_list[L - 1], (1, 0))
    starts = jnp.searchsorted(dbatch, jnp.arange(NBATCH), side="left")
    ends = jnp.searchsorted(dbatch, jnp.arange(NBATCH), side="right")
    row_start = starts[dbatch].astype(jnp.int32).reshape(1, n)
    row_end = ends[dbatch].astype(jnp.int32).reshape(1, n)

    u = jax.random.uniform(jax.random.fold_in(jax.random.key(0), 7),
                           (n, NFR * K), minval=1e-10, maxval=1.0)
    gT = jnp.transpose(-jnp.log(-jnp.log(u)), (1, 0))
    delta_t = jnp.repeat(jnp.arange(NFR, 0, -1), K).astype(jnp.float32)
    prtT = jax.nn.softmax(delta_t * -0.5).reshape(NFR * K, 1)

    drT, idxT = _run_knn(anchT, xyz1_list[:NFR], row_start, row_end, n)

    drT_scr = jnp.transpose(
        jnp.transpose(drT, (0, 2, 1)).reshape(n, NFR * K), (1, 0))
    idxT_scr = jnp.transpose(
        jnp.transpose(idxT, (0, 2, 1)).reshape(n, NFR * K), (1, 0))
    ridT = _run_score(drT_scr, idxT_scr, gT, prtT, n)
    rid = jnp.transpose(ridT, (1, 0))

    f_all = f1_list.reshape(L * n, h)
    proj = _run_proj(f_all, W1, b1.reshape(1, h), n, h)
    table = proj[:NFR * n]
    a_proj = proj[NFR * n:]

    b_flat = _run_sc_gather(table, rid.reshape(n * NS), n, h)
    b_rows = b_flat.reshape(n, NS, h)

    return _run_tail(a_proj, b_rows, W2, b2.reshape(1, h), n, h)


# speedup vs baseline: 1.0580x; 1.0580x over previous
import functools

import jax
import jax.numpy as jnp
from jax import lax
from jax.experimental import pallas as pl
from jax.experimental.pallas import tpu as pltpu
from jax.experimental.pallas import tpu_sc as plsc

K = 16
LF = 4
NFR = LF - 1
NS = 24
NBATCH = 8
RSC = 256
RMLP = 256
CH = 384
INT_BIG = 2**31 - 1


AB = 128
CHK = 16


def _xor_perm(x, j, ri):
    up = jnp.roll(x, -j, axis=0)
    dn = jnp.roll(x, j, axis=0)
    return jnp.where((ri & j) == 0, up, dn)


def _bitonic_stage(v, ix, j, take_min, ri):
    pv = _xor_perm(v, j, ri)
    pi = _xor_perm(ix, j, ri)
    swap = v > pv
    mnv = jnp.where(swap, pv, v)
    mni = jnp.where(swap, pi, ix)
    mxv = jnp.where(swap, v, pv)
    mxi = jnp.where(swap, ix, pi)
    return (jnp.where(take_min, mnv, mxv),
            jnp.where(take_min, mni, mxi))


def _sort_desc(v, ix, ri, size):
    k = 2
    while k <= size:
        upmask = (ri & k) == 0
        j = k // 2
        while j >= 1:
            jm = (ri & j) == 0
            take_min = ~(jm == upmask)
            v, ix = _bitonic_stage(v, ix, j, take_min, ri)
            j //= 2
        k *= 2
    return v, ix


NHALF = 2
ABW = AB * NHALF


def _knn_body(anch_ref, cand_ref, rs_ref, re_ref, dr_ref, idx_ref):
    ax, ay, az, rs, re = [], [], [], [], []
    for hb in range(NHALF):
        sl = slice(hb * AB, (hb + 1) * AB)
        ax.append(anch_ref[0:1, sl])
        ay.append(anch_ref[1:2, sl])
        az.append(anch_ref[2:3, sl])
        rs.append(rs_ref[0:1, sl])
        re.append(re_ref[0:1, sl])
    lo = jnp.min(rs_ref[...]) // CHK
    hi = (jnp.max(re_ref[...]) + CHK - 1) // CHK
    ri = lax.broadcasted_iota(jnp.int32, (CHK, 1), 0)

    sv0 = jnp.full((CHK, AB), jnp.inf, dtype=jnp.float32)
    si0 = jnp.broadcast_to(2**30 + ri, (CHK, AB))

    def chunk_step(c, carry):
        base = c * CHK
        row = base + ri
        out = []
        for f in range(NFR):
            ch = cand_ref[f, pl.ds(base, CHK), :]
            cx = ch[:, 0:1]
            cy = ch[:, 1:2]
            cz = ch[:, 2:3]
            for hb in range(NHALF):
                sv, si = carry[2 * (NHALF * f + hb)], \
                    carry[2 * (NHALF * f + hb) + 1]
                dx = cx - ax[hb]
                dy = cy - ay[hb]
                dz = cz - az[hb]
                d = dx * dx + dy * dy + dz * dz
                valid = (row >= rs[hb]) & (row < re[hb])
                d = jnp.where(valid, d, jnp.inf)
                cix = jnp.broadcast_to(row, (CHK, AB))
                cv, cix = _sort_desc(d, cix, ri, CHK)
                swap = sv > cv
                v = jnp.where(swap, cv, sv)
                ix = jnp.where(swap, cix, si)
                for j in (8, 4, 2, 1):
                    v, ix = _bitonic_stage(v, ix, j, (ri & j) == 0, ri)
                out += [v, ix]
        return tuple(out)

    init = (sv0, si0) * (NFR * NHALF)
    res = lax.fori_loop(lo, hi, chunk_step, init)
    for f in range(NFR):
        dr_ref[f] = jnp.sqrt(jnp.concatenate(
            [res[2 * (NHALF * f + hb)] for hb in range(NHALF)], axis=1))
        idx_ref[f] = jnp.concatenate(
            [res[2 * (NHALF * f + hb) + 1] for hb in range(NHALF)], axis=1)


def _run_knn(anchT, xyz_frames, row_start1, row_end1, n):
    grid = (n // ABW,)
    return pl.pallas_call(
        _knn_body,
        grid=grid,
        in_specs=[
            pl.BlockSpec((3, ABW), lambda ab: (0, ab)),
            pl.BlockSpec((NFR, n, 3), lambda ab: (0, 0, 0)),
            pl.BlockSpec((1, ABW), lambda ab: (0, ab)),
            pl.BlockSpec((1, ABW), lambda ab: (0, ab)),
        ],
        out_specs=[
            pl.BlockSpec((NFR, CHK, ABW), lambda ab: (0, 0, ab)),
            pl.BlockSpec((NFR, CHK, ABW), lambda ab: (0, 0, ab)),
        ],
        out_shape=[
            jax.ShapeDtypeStruct((NFR, CHK, n), jnp.float32),
            jax.ShapeDtypeStruct((NFR, CHK, n), jnp.int32),
        ],
    )(anchT, xyz_frames, row_start1, row_end1)


def _score_body(dr_ref, idx_ref, g_ref, prt_ref, rid_ref, *, n):
    cols = NFR * K
    x = dr_ref[...] * (-0.5)
    m = jnp.max(x, axis=0, keepdims=True)
    e = jnp.exp(x - m)
    pr_r = e / jnp.sum(e, axis=0, keepdims=True)
    p = 0.5 * pr_r + 0.5 * prt_ref[...]
    sc = jnp.log(p + 1e-12) + g_ref[...]

    pid = pl.program_id(0)
    ri48 = lax.broadcasted_iota(jnp.int32, (cols, 1), 0)
    li = lax.broadcasted_iota(jnp.int32, (1, AB), 1)
    pos = cols * (pid * AB + li) + ri48
    frame = pos // (n * K)
    nbg = idx_ref[...] + frame * n
    nbg = jnp.minimum(nbg, jnp.int32(NFR * n - 1))
    comb = ri48 * 32768 + nbg

    pad = 64 - cols
    scp = jnp.concatenate(
        [sc, jnp.full((pad, AB), -jnp.inf, dtype=jnp.float32)], axis=0)
    cbp = jnp.concatenate(
        [comb, jnp.full((pad, AB), INT_BIG, dtype=jnp.int32)], axis=0)
    ri64 = lax.broadcasted_iota(jnp.int32, (64, 1), 0)
    scp, cbp = _sort_desc(scp, cbp, ri64, 64)
    rid_ref[...] = cbp[0:NS, :] & 32767


def _run_score(drT_scr, idxT_scr, gT, prtT, n):
    cols = NFR * K
    grid = (n // AB,)
    return pl.pallas_call(
        functools.partial(_score_body, n=n),
        grid=grid,
        in_specs=[
            pl.BlockSpec((cols, AB), lambda rb: (0, rb)),
            pl.BlockSpec((cols, AB), lambda rb: (0, rb)),
            pl.BlockSpec((cols, AB), lambda rb: (0, rb)),
            pl.BlockSpec((cols, 1), lambda rb: (0, 0)),
        ],
        out_specs=pl.BlockSpec((NS, AB), lambda rb: (0, rb)),
        out_shape=jax.ShapeDtypeStruct((NS, n), jnp.int32),
    )(drT_scr, idxT_scr, gT, prtT)


def _proj_body(f_ref, w1_ref, b1_ref, out_ref, *, h, nblk_src):
    pid = pl.program_id(0)
    w1a = w1_ref[0:h, :]
    w1b = w1_ref[h:2 * h, :]

    @pl.when(pid < nblk_src)
    def _():
        out_ref[...] = jnp.dot(f_ref[...], w1b,
                               preferred_element_type=jnp.float32)

    @pl.when(pid >= nblk_src)
    def _():
        out_ref[...] = (jnp.dot(f_ref[...], w1a + w1b,
                                preferred_element_type=jnp.float32)
                        + b1_ref[...])


def _run_proj(f_all, w1, b1_row, n, h):
    rows = LF * n
    blk = 512
    nblk_src = (NFR * n) // blk
    return pl.pallas_call(
        functools.partial(_proj_body, h=h, nblk_src=nblk_src),
        grid=(rows // blk,),
        in_specs=[
            pl.BlockSpec((blk, h), lambda rb: (rb, 0)),
            pl.BlockSpec((2 * h, h), lambda rb: (0, 0)),
            pl.BlockSpec((1, h), lambda rb: (0, 0)),
        ],
        out_specs=pl.BlockSpec((blk, h), lambda rb: (rb, 0)),
        out_shape=jax.ShapeDtypeStruct((rows, h), jnp.float32),
    )(f_all, w1, b1_row)


def _run_sc_gather(table, rid_flat, n, h):
    total = n * NS
    info = plsc.get_sparse_core_info()
    nw = info.num_cores * info.num_subcores
    b_per_w = total // nw
    nch = b_per_w // CH
    mesh = plsc.VectorSubcoreMesh(core_axis_name="c", subcore_axis_name="s")

    @functools.partial(
        pl.kernel,
        mesh=mesh,
        out_type=jax.ShapeDtypeStruct((total, h), jnp.float32),
        scratch_types=[
            pltpu.VMEM((CH,), jnp.int32),
            pltpu.VMEM((CH,), jnp.int32),
            pltpu.VMEM((CH, h), jnp.float32),
            pltpu.VMEM((CH, h), jnp.float32),
            pltpu.SemaphoreType.DMA,
            pltpu.SemaphoreType.DMA,
        ],
    )
    def gather_k(table_hbm, idx_hbm, out_hbm, idx_v0, idx_v1,
                 rows_v0, rows_v1, sem0, sem1):
        wid = lax.axis_index("s") * info.num_cores + lax.axis_index("c")
        base = wid * b_per_w
        idxs = [idx_v0, idx_v1]
        rows = [rows_v0, rows_v1]
        sems = [sem0, sem1]

        pltpu.sync_copy(idx_hbm.at[pl.ds(base, CH)], idxs[0])
        handles = [pltpu.async_copy(table_hbm.at[idxs[0]], rows[0], sems[0]),
                   None]
        for c in range(nch):
            b = c % 2
            nb = (c + 1) % 2
            if c + 1 < nch:
                off_n = base + (c + 1) * CH
                pltpu.sync_copy(idx_hbm.at[pl.ds(off_n, CH)], idxs[nb])
                handles[nb] = pltpu.async_copy(
                    table_hbm.at[idxs[nb]], rows[nb], sems[nb])
            handles[b].wait()
            pltpu.sync_copy(rows[b], out_hbm.at[pl.ds(base + c * CH, CH)])

    return gather_k(table, rid_flat)


def _tail_body(a_ref, b_ref, w2_ref, b2_ref, out_ref):
    a = a_ref[...]
    w2 = w2_ref[...]
    acc = None
    for s in range(NS):
        hsd = jnp.maximum(a - b_ref[:, s, :], 0.0)
        o = jnp.dot(hsd, w2, preferred_element_type=jnp.float32)
        acc = o if acc is None else jnp.maximum(acc, o)
    out_ref[...] = jnp.tanh(acc + b2_ref[...])


def _run_tail(a_proj, b_rows, w2, b2_row, n, h):
    return pl.pallas_call(
        _tail_body,
        grid=(n // RMLP,),
        in_specs=[
            pl.BlockSpec((RMLP, h), lambda rb: (rb, 0)),
            pl.BlockSpec((RMLP, NS, h), lambda rb: (rb, 0, 0)),
            pl.BlockSpec((h, h), lambda rb: (0, 0)),
            pl.BlockSpec((1, h), lambda rb: (0, 0)),
        ],
        out_specs=pl.BlockSpec((RMLP, h), lambda rb: (rb, 0)),
        out_shape=jax.ShapeDtypeStruct((n, h), jnp.float32),
    )(a_proj, b_rows, w2, b2_row)


def kernel(num_samples, f1_list, xyz1_list, dbatch, W1, b1, W2, b2):
    L, n, h = f1_list.shape
    del num_samples

    anchT = jnp.transpose(xyz1_list[L - 1], (1, 0))
    starts = jnp.searchsorted(dbatch, jnp.arange(NBATCH), side="left")
    ends = jnp.searchsorted(dbatch, jnp.arange(NBATCH), side="right")
    row_start = starts[dbatch].astype(jnp.int32).reshape(1, n)
    row_end = ends[dbatch].astype(jnp.int32).reshape(1, n)

    u = jax.random.uniform(jax.random.fold_in(jax.random.key(0), 7),
                           (n, NFR * K), minval=1e-10, maxval=1.0)
    gT = jnp.transpose(-jnp.log(-jnp.log(u)), (1, 0))
    delta_t = jnp.repeat(jnp.arange(NFR, 0, -1), K).astype(jnp.float32)
    prtT = jax.nn.softmax(delta_t * -0.5).reshape(NFR * K, 1)

    drT, idxT = _run_knn(anchT, xyz1_list[:NFR], row_start, row_end, n)

    drT_scr = jnp.transpose(
        jnp.transpose(drT, (0, 2, 1)).reshape(n, NFR * K), (1, 0))
    idxT_scr = jnp.transpose(
        jnp.transpose(idxT, (0, 2, 1)).reshape(n, NFR * K), (1, 0))
    ridT = _run_score(drT_scr, idxT_scr, gT, prtT, n)
    rid = jnp.transpose(ridT, (1, 0))

    f_all = f1_list.reshape(L * n, h)
    proj = _run_proj(f_all, W1, b1.reshape(1, h), n, h)
    table = proj[:NFR * n]
    a_proj = proj[NFR * n:]

    b_flat = _run_sc_gather(table, rid.reshape(n * NS), n, h)
    b_rows = b_flat.reshape(n, NS, h)

    return _run_tail(a_proj, b_rows, W2, b2.reshape(1, h), n, h)
